# baseline (speedup 1.0000x reference)
import functools

import jax
import jax.numpy as jnp
from jax import lax
from jax.experimental import pallas as pl
from jax.experimental.pallas import tpu as pltpu
from jax.experimental.pallas import tpu_sc as plsc

N = 10000
E = 320000
D = 128

NC = 2
NS = 16
NW = NC * NS
CHUNK = 128
N_CHUNKS = 160
E_PAD = NS * CHUNK * N_CHUNKS
TOTAL_CHUNKS = E_PAD // CHUNK
ROWS_PER_SUB = 624
ROWS_TAIL = N - NS * ROWS_PER_SUB
PAD_ROWS = 512
ACC_ROWS = N + PAD_ROWS

ROW_BLK = 1000


def _h_body(x_ref, p_ref, wq_ref, bq_ref, wk_ref, bk_ref, h_ref):
    q = jnp.dot(x_ref[...], wq_ref[...],
                preferred_element_type=jnp.float32) + bq_ref[...]
    k = jnp.dot(p_ref[...], wk_ref[...],
                preferred_element_type=jnp.float32) + bk_ref[...]
    h_ref[...] = q * k


def _compute_h(x, prompt, WqT, bq, WkT, bk):
    grid = (N // ROW_BLK,)
    row_spec = pl.BlockSpec((ROW_BLK, D), lambda i: (i, 0))
    w_spec = pl.BlockSpec((D, D), lambda i: (0, 0))
    b_spec = pl.BlockSpec((1, D), lambda i: (0, 0))
    return pl.pallas_call(
        _h_body,
        grid=grid,
        in_specs=[row_spec, row_spec, w_spec, b_spec, w_spec, b_spec],
        out_specs=row_spec,
        out_shape=jax.ShapeDtypeStruct((N, D), jnp.float32),
    )(x, prompt, WqT, bq, WkT, bk)


def _sc_seg_sum(h, ei, zeros):
    mesh = plsc.VectorSubcoreMesh(core_axis_name="c", subcore_axis_name="s")

    @functools.partial(
        pl.kernel,
        out_type=jax.ShapeDtypeStruct((N, D), jnp.float32),
        mesh=mesh,
        scratch_types=[
            pltpu.VMEM((2, CHUNK), jnp.int32),
            pltpu.VMEM((2, CHUNK), jnp.int32),
            pltpu.VMEM((2, CHUNK, D), jnp.float32),
            pltpu.VMEM_SHARED((ACC_ROWS, D), jnp.float32),
            pltpu.SemaphoreType.DMA,
            pltpu.SemaphoreType.DMA,
            pltpu.SemaphoreType.DMA,
            pltpu.SemaphoreType.DMA,
        ],
    )
    def seg_sum(h_hbm, ei_hbm, z_hbm, out_hbm,
                idx_a, idx_b, rows, accum, sem_a, sem_b, sem_sa, sem_sb):
        cid = lax.axis_index("c")
        sid = lax.axis_index("s")

        @pl.when(cid == 0)
        def _():
            base_c = sid * N_CHUNKS
            n_iter = N_CHUNKS // 2

            r0 = sid * ROWS_PER_SUB
            pltpu.sync_copy(z_hbm.at[pl.ds(r0, ROWS_PER_SUB)],
                            accum.at[pl.ds(r0, ROWS_PER_SUB)])

            @pl.when(sid == 0)
            def _():
                pltpu.sync_copy(z_hbm.at[pl.ds(NS * ROWS_PER_SUB, ROWS_TAIL)],
                                accum.at[pl.ds(NS * ROWS_PER_SUB, ROWS_TAIL)])

            plsc.subcore_barrier()

            rows_a = rows.at[0]
            rows_b = rows.at[1]

            pltpu.sync_copy(ei_hbm.at[base_c], idx_a)
            pltpu.async_copy(h_hbm.at[idx_a.at[0]], rows_a, sem_a)
            pltpu.sync_copy(ei_hbm.at[base_c + 1], idx_b)
            pltpu.async_copy(h_hbm.at[idx_b.at[0]], rows_b, sem_b)

            @pl.loop(0, n_iter)
            def _(j):
                i0 = base_c + 2 * j
                pltpu.make_async_copy(h_hbm.at[idx_a.at[0]], rows_a,
                                      sem_a).wait()
                pltpu.sync_copy(rows_a, accum.at[idx_a.at[1]], add=True)

                @pl.when(j < n_iter - 1)
                def _():
                    pltpu.sync_copy(ei_hbm.at[i0 + 2], idx_a)
                    pltpu.async_copy(h_hbm.at[idx_a.at[0]], rows_a, sem_a)

                pltpu.make_async_copy(h_hbm.at[idx_b.at[0]], rows_b,
                                      sem_b).wait()
                pltpu.sync_copy(rows_b, accum.at[idx_b.at[1]], add=True)

                @pl.when(j < n_iter - 1)
                def _():
                    pltpu.sync_copy(ei_hbm.at[i0 + 3], idx_b)
                    pltpu.async_copy(h_hbm.at[idx_b.at[0]], rows_b, sem_b)

            plsc.subcore_barrier()
            pltpu.sync_copy(accum.at[pl.ds(r0, ROWS_PER_SUB)],
                            out_hbm.at[pl.ds(r0, ROWS_PER_SUB)])

            @pl.when(sid == 0)
            def _():
                pltpu.sync_copy(accum.at[pl.ds(NS * ROWS_PER_SUB, ROWS_TAIL)],
                                out_hbm.at[pl.ds(NS * ROWS_PER_SUB,
                                                 ROWS_TAIL)])

    return seg_sum(h, ei, zeros)


def _out_body(h_ref, p_ref, a0_ref, x_ref,
              wv_ref, bv_ref, wh_ref, bh_ref,
              w1_ref, b1_ref, w2_ref, b2_ref, o_ref):
    v = jnp.dot(p_ref[...], wv_ref[...],
                preferred_element_type=jnp.float32) + bv_ref[...]
    t = h_ref[...] + v + a0_ref[...]
    t = jnp.dot(t, wh_ref[...],
                preferred_element_type=jnp.float32) + bh_ref[...] + x_ref[...]
    t = jnp.dot(t, w1_ref[...],
                preferred_element_type=jnp.float32) + b1_ref[...]
    o_ref[...] = jnp.dot(t, w2_ref[...],
                         preferred_element_type=jnp.float32) + b2_ref[...]


def _compute_out(h, prompt, a0, x, WvT, bv, WhT, bh, W1T, b1, W2T, b2):
    grid = (N // ROW_BLK,)
    row_spec = pl.BlockSpec((ROW_BLK, D), lambda i: (i, 0))
    wdd_spec = pl.BlockSpec((D, D), lambda i: (0, 0))
    bd_spec = pl.BlockSpec((1, D), lambda i: (0, 0))
    w1_spec = pl.BlockSpec((D, 2 * D), lambda i: (0, 0))
    b1_spec = pl.BlockSpec((1, 2 * D), lambda i: (0, 0))
    w2_spec = pl.BlockSpec((2 * D, D), lambda i: (0, 0))
    return pl.pallas_call(
        _out_body,
        grid=grid,
        in_specs=[row_spec, row_spec, row_spec, row_spec,
                  wdd_spec, bd_spec, wdd_spec, bd_spec,
                  w1_spec, b1_spec, w2_spec, bd_spec],
        out_specs=row_spec,
        out_shape=jax.ShapeDtypeStruct((N, D), jnp.float32),
    )(h, prompt, a0, x, WvT, bv, WhT, bh, W1T, b1, W2T, b2)


def kernel(x, edge_index, prompt, Wq, bq, Wk, bk, Wv, bv, Wh, bh,
           W1, b1, W2, b2):
    src = edge_index[0]
    dst = edge_index[1]
    pad = E_PAD - E
    src_p = jnp.concatenate([src, jnp.zeros((pad,), jnp.int32)])
    dst_pad = N + (jnp.arange(pad, dtype=jnp.int32) % PAD_ROWS)
    dst_p = jnp.concatenate([dst, dst_pad])
    ei_p = jnp.stack([src_p.reshape(TOTAL_CHUNKS, CHUNK),
                      dst_p.reshape(TOTAL_CHUNKS, CHUNK)], axis=1)

    h = _compute_h(x, prompt, Wq.T, bq.reshape(1, D), Wk.T, bk.reshape(1, D))

    zeros = jnp.zeros((N, D), jnp.float32)
    aggr = _sc_seg_sum(h, ei_p, zeros)

    out = _compute_out(h, prompt, aggr, x,
                       Wv.T, bv.reshape(1, D), Wh.T, bh.reshape(1, D),
                       W1.T, b1.reshape(1, 2 * D), W2.T, b2.reshape(1, D))
    return out


# speedup vs baseline: 2.6847x; 2.6847x over previous
import functools

import jax
import jax.numpy as jnp
from jax import lax
from jax.experimental import pallas as pl
from jax.experimental.pallas import tpu as pltpu
from jax.experimental.pallas import tpu_sc as plsc

N = 10000
E = 320000
D = 128

NC = 2
NS = 16
NW = NC * NS
CHUNK = 128
N_CHUNKS = 80
E_PAD = NW * CHUNK * N_CHUNKS
TOTAL_CHUNKS = E_PAD // CHUNK
ROWS_PER_SUB = 624
ROWS_TAIL = N - NS * ROWS_PER_SUB
PAD_ROWS = 512
ACC_ROWS = N + PAD_ROWS

ROW_BLK = 1000


def _h_body(x_ref, p_ref, wq_ref, bq_ref, wk_ref, bk_ref, h_ref):
    q = jnp.dot(x_ref[...], wq_ref[...],
                preferred_element_type=jnp.float32) + bq_ref[...]
    k = jnp.dot(p_ref[...], wk_ref[...],
                preferred_element_type=jnp.float32) + bk_ref[...]
    h_ref[...] = q * k


def _compute_h(x, prompt, WqT, bq, WkT, bk):
    grid = (N // ROW_BLK,)
    row_spec = pl.BlockSpec((ROW_BLK, D), lambda i: (i, 0))
    w_spec = pl.BlockSpec((D, D), lambda i: (0, 0))
    b_spec = pl.BlockSpec((1, D), lambda i: (0, 0))
    return pl.pallas_call(
        _h_body,
        grid=grid,
        in_specs=[row_spec, row_spec, w_spec, b_spec, w_spec, b_spec],
        out_specs=row_spec,
        out_shape=jax.ShapeDtypeStruct((N, D), jnp.float32),
    )(x, prompt, WqT, bq, WkT, bk)


def _sc_seg_sum(h, ei, zeros):
    mesh = plsc.VectorSubcoreMesh(core_axis_name="c", subcore_axis_name="s")

    @functools.partial(
        pl.kernel,
        out_type=jax.ShapeDtypeStruct((NC, N, D), jnp.float32),
        mesh=mesh,
        scratch_types=[
            pltpu.VMEM((2, CHUNK), jnp.int32),
            pltpu.VMEM((2, CHUNK), jnp.int32),
            pltpu.VMEM((2, CHUNK, D), jnp.float32),
            pltpu.VMEM_SHARED((ACC_ROWS, D), jnp.float32),
            pltpu.SemaphoreType.DMA,
            pltpu.SemaphoreType.DMA,
            pltpu.SemaphoreType.DMA,
            pltpu.SemaphoreType.DMA,
        ],
    )
    def seg_sum(h_hbm, ei_hbm, z_hbm, out_hbm,
                idx_a, idx_b, rows, accum, sem_a, sem_b, sem_sa, sem_sb):
        cid = lax.axis_index("c")
        sid = lax.axis_index("s")
        wid = cid * NS + sid
        base_c = wid * N_CHUNKS
        n_iter = N_CHUNKS // 2

        r0 = sid * ROWS_PER_SUB
        pltpu.sync_copy(z_hbm.at[pl.ds(r0, ROWS_PER_SUB)],
                        accum.at[pl.ds(r0, ROWS_PER_SUB)])

        @pl.when(sid == 0)
        def _():
            pltpu.sync_copy(z_hbm.at[pl.ds(NS * ROWS_PER_SUB, ROWS_TAIL)],
                            accum.at[pl.ds(NS * ROWS_PER_SUB, ROWS_TAIL)])

        plsc.subcore_barrier()

        rows_a = rows.at[0]
        rows_b = rows.at[1]

        pltpu.sync_copy(ei_hbm.at[base_c], idx_a)
        pltpu.async_copy(h_hbm.at[idx_a.at[0]], rows_a, sem_a)
        pltpu.sync_copy(ei_hbm.at[base_c + 1], idx_b)
        pltpu.async_copy(h_hbm.at[idx_b.at[0]], rows_b, sem_b)

        @pl.loop(0, n_iter)
        def _(j):
            i0 = base_c + 2 * j
            pltpu.make_async_copy(h_hbm.at[idx_a.at[0]], rows_a, sem_a).wait()
            pltpu.sync_copy(rows_a, accum.at[idx_a.at[1]], add=True)

            @pl.when(j < n_iter - 1)
            def _():
                pltpu.sync_copy(ei_hbm.at[i0 + 2], idx_a)
                pltpu.async_copy(h_hbm.at[idx_a.at[0]], rows_a, sem_a)

            pltpu.make_async_copy(h_hbm.at[idx_b.at[0]], rows_b, sem_b).wait()
            pltpu.sync_copy(rows_b, accum.at[idx_b.at[1]], add=True)

            @pl.when(j < n_iter - 1)
            def _():
                pltpu.sync_copy(ei_hbm.at[i0 + 3], idx_b)
                pltpu.async_copy(h_hbm.at[idx_b.at[0]], rows_b, sem_b)

        plsc.subcore_barrier()
        pltpu.sync_copy(accum.at[pl.ds(r0, ROWS_PER_SUB)],
                        out_hbm.at[cid, pl.ds(r0, ROWS_PER_SUB)])

        @pl.when(sid == 0)
        def _():
            pltpu.sync_copy(accum.at[pl.ds(NS * ROWS_PER_SUB, ROWS_TAIL)],
                            out_hbm.at[cid, pl.ds(NS * ROWS_PER_SUB,
                                                  ROWS_TAIL)])

    return seg_sum(h, ei, zeros)


def _out_body(h_ref, p_ref, a0_ref, a1_ref, x_ref,
              wv_ref, bv_ref, wh_ref, bh_ref,
              w1_ref, b1_ref, w2_ref, b2_ref, o_ref):
    v = jnp.dot(p_ref[...], wv_ref[...],
                preferred_element_type=jnp.float32) + bv_ref[...]
    t = h_ref[...] + v + a0_ref[...] + a1_ref[...]
    t = jnp.dot(t, wh_ref[...],
                preferred_element_type=jnp.float32) + bh_ref[...] + x_ref[...]
    t = jnp.dot(t, w1_ref[...],
                preferred_element_type=jnp.float32) + b1_ref[...]
    o_ref[...] = jnp.dot(t, w2_ref[...],
                         preferred_element_type=jnp.float32) + b2_ref[...]


def _compute_out(h, prompt, a0, a1, x, WvT, bv, WhT, bh, W1T, b1, W2T, b2):
    grid = (N // ROW_BLK,)
    row_spec = pl.BlockSpec((ROW_BLK, D), lambda i: (i, 0))
    wdd_spec = pl.BlockSpec((D, D), lambda i: (0, 0))
    bd_spec = pl.BlockSpec((1, D), lambda i: (0, 0))
    w1_spec = pl.BlockSpec((D, 2 * D), lambda i: (0, 0))
    b1_spec = pl.BlockSpec((1, 2 * D), lambda i: (0, 0))
    w2_spec = pl.BlockSpec((2 * D, D), lambda i: (0, 0))
    return pl.pallas_call(
        _out_body,
        grid=grid,
        in_specs=[row_spec, row_spec, row_spec, row_spec, row_spec,
                  wdd_spec, bd_spec, wdd_spec, bd_spec,
                  w1_spec, b1_spec, w2_spec, bd_spec],
        out_specs=row_spec,
        out_shape=jax.ShapeDtypeStruct((N, D), jnp.float32),
    )(h, prompt, a0, a1, x, WvT, bv, WhT, bh, W1T, b1, W2T, b2)


def kernel(x, edge_index, prompt, Wq, bq, Wk, bk, Wv, bv, Wh, bh,
           W1, b1, W2, b2):
    src = edge_index[0]
    dst = edge_index[1]
    pad = E_PAD - E
    pad_iota = jnp.arange(pad, dtype=jnp.int32)
    src_p = jnp.concatenate([src, pad_iota % N])
    dst_p = jnp.concatenate([dst, N + (pad_iota % PAD_ROWS)])
    ei_p = jnp.stack([src_p.reshape(TOTAL_CHUNKS, CHUNK),
                      dst_p.reshape(TOTAL_CHUNKS, CHUNK)], axis=1)

    h = _compute_h(x, prompt, Wq.T, bq.reshape(1, D), Wk.T, bk.reshape(1, D))

    zeros = jnp.zeros((N, D), jnp.float32)
    aggr = _sc_seg_sum(h, ei_p, zeros)

    out = _compute_out(h, prompt, aggr[0], aggr[1], x,
                       Wv.T, bv.reshape(1, D), Wh.T, bh.reshape(1, D),
                       W1.T, b1.reshape(1, 2 * D), W2.T, b2.reshape(1, D))
    return out


# speedup vs baseline: 2.9761x; 1.1085x over previous
import functools

import jax
import jax.numpy as jnp
from jax import lax
from jax.experimental import pallas as pl
from jax.experimental.pallas import tpu as pltpu
from jax.experimental.pallas import tpu_sc as plsc

N = 10000
E = 320000
D = 128

NC = 2
NS = 16
NW = NC * NS
CHUNK = 128
N_CHUNKS = 80
E_PAD = NW * CHUNK * N_CHUNKS
TOTAL_CHUNKS = E_PAD // CHUNK
ROWS_PER_SUB = 624
ROWS_TAIL = N - NS * ROWS_PER_SUB
PAD_ROWS = 512
ACC_ROWS = N + PAD_ROWS

ROW_BLK = 1000
EBLK = E_PAD // 10
CBLK = EBLK // CHUNK


def _dot_t(a, w):
    return lax.dot_general(a, w, (((1,), (1,)), ((), ())),
                           preferred_element_type=jnp.float32)


def _prep_body(ei_ref, o_ref):
    i = pl.program_id(0)
    row_i = lax.broadcasted_iota(jnp.int32, (CBLK, CHUNK), 0)
    col_i = lax.broadcasted_iota(jnp.int32, (CBLK, CHUNK), 1)
    p = i * EBLK + row_i * CHUNK + col_i
    mask = p < E
    q = p - E
    s = ei_ref[0].reshape(CBLK, CHUNK)
    d = ei_ref[1].reshape(CBLK, CHUNK)
    o_ref[:, 0, :] = jnp.where(mask, s, q % N)
    o_ref[:, 1, :] = jnp.where(mask, d, N + (q % PAD_ROWS))


def _prep_edges(edge_index):
    return pl.pallas_call(
        _prep_body,
        grid=(E_PAD // EBLK,),
        in_specs=[pl.BlockSpec((2, EBLK), lambda i: (0, i))],
        out_specs=pl.BlockSpec((CBLK, 2, CHUNK), lambda i: (i, 0, 0)),
        out_shape=jax.ShapeDtypeStruct((TOTAL_CHUNKS, 2, CHUNK), jnp.int32),
    )(edge_index)


def _h_body(x_ref, p_ref, wq_ref, bq_ref, wk_ref, bk_ref, wv_ref, bv_ref,
            h_ref, s_ref, z_ref):
    q = _dot_t(x_ref[...], wq_ref[...]) + bq_ref[...]
    k = _dot_t(p_ref[...], wk_ref[...]) + bk_ref[...]
    v = _dot_t(p_ref[...], wv_ref[...]) + bv_ref[...]
    h = q * k
    h_ref[...] = h
    s_ref[...] = h + v
    z_ref[...] = jnp.zeros_like(z_ref)


def _compute_h(x, prompt, Wq, bq, Wk, bk, Wv, bv):
    grid = (N // ROW_BLK,)
    row_spec = pl.BlockSpec((ROW_BLK, D), lambda i: (i, 0))
    w_spec = pl.BlockSpec((D, D), lambda i: (0, 0))
    b_spec = pl.BlockSpec((1, D), lambda i: (0, 0))
    return pl.pallas_call(
        _h_body,
        grid=grid,
        in_specs=[row_spec, row_spec, w_spec, b_spec, w_spec, b_spec,
                  w_spec, b_spec],
        out_specs=[row_spec, row_spec, row_spec],
        out_shape=[jax.ShapeDtypeStruct((N, D), jnp.float32),
                   jax.ShapeDtypeStruct((N, D), jnp.float32),
                   jax.ShapeDtypeStruct((N, D), jnp.float32)],
    )(x, prompt, Wq, bq, Wk, bk, Wv, bv)


def _sc_seg_sum(h, ei, zeros):
    mesh = plsc.VectorSubcoreMesh(core_axis_name="c", subcore_axis_name="s")

    @functools.partial(
        pl.kernel,
        out_type=jax.ShapeDtypeStruct((NC, N, D), jnp.float32),
        mesh=mesh,
        scratch_types=[
            pltpu.VMEM((2, CHUNK), jnp.int32),
            pltpu.VMEM((2, CHUNK), jnp.int32),
            pltpu.VMEM((2, CHUNK, D), jnp.float32),
            pltpu.VMEM_SHARED((ACC_ROWS, D), jnp.float32),
            pltpu.SemaphoreType.DMA,
            pltpu.SemaphoreType.DMA,
        ],
    )
    def seg_sum(h_hbm, ei_hbm, z_hbm, out_hbm,
                idx_a, idx_b, rows, accum, sem_a, sem_b):
        cid = lax.axis_index("c")
        sid = lax.axis_index("s")
        wid = cid * NS + sid
        base_c = wid * N_CHUNKS
        n_iter = N_CHUNKS // 2

        r0 = sid * ROWS_PER_SUB
        pltpu.sync_copy(z_hbm.at[pl.ds(r0, ROWS_PER_SUB)],
                        accum.at[pl.ds(r0, ROWS_PER_SUB)])

        @pl.when(sid == 0)
        def _():
            pltpu.sync_copy(z_hbm.at[pl.ds(NS * ROWS_PER_SUB, ROWS_TAIL)],
                            accum.at[pl.ds(NS * ROWS_PER_SUB, ROWS_TAIL)])

        plsc.subcore_barrier()

        rows_a = rows.at[0]
        rows_b = rows.at[1]

        pltpu.sync_copy(ei_hbm.at[base_c], idx_a)
        pltpu.async_copy(h_hbm.at[idx_a.at[0]], rows_a, sem_a)
        pltpu.sync_copy(ei_hbm.at[base_c + 1], idx_b)
        pltpu.async_copy(h_hbm.at[idx_b.at[0]], rows_b, sem_b)

        @pl.loop(0, n_iter)
        def _(j):
            i0 = base_c + 2 * j
            pltpu.make_async_copy(h_hbm.at[idx_a.at[0]], rows_a, sem_a).wait()
            pltpu.sync_copy(rows_a, accum.at[idx_a.at[1]], add=True)

            @pl.when(j < n_iter - 1)
            def _():
                pltpu.sync_copy(ei_hbm.at[i0 + 2], idx_a)
                pltpu.async_copy(h_hbm.at[idx_a.at[0]], rows_a, sem_a)

            pltpu.make_async_copy(h_hbm.at[idx_b.at[0]], rows_b, sem_b).wait()
            pltpu.sync_copy(rows_b, accum.at[idx_b.at[1]], add=True)

            @pl.when(j < n_iter - 1)
            def _():
                pltpu.sync_copy(ei_hbm.at[i0 + 3], idx_b)
                pltpu.async_copy(h_hbm.at[idx_b.at[0]], rows_b, sem_b)

        plsc.subcore_barrier()
        pltpu.sync_copy(accum.at[pl.ds(r0, ROWS_PER_SUB)],
                        out_hbm.at[cid, pl.ds(r0, ROWS_PER_SUB)])

        @pl.when(sid == 0)
        def _():
            pltpu.sync_copy(accum.at[pl.ds(NS * ROWS_PER_SUB, ROWS_TAIL)],
                            out_hbm.at[cid, pl.ds(NS * ROWS_PER_SUB,
                                                  ROWS_TAIL)])

    return seg_sum(h, ei, zeros)


def _out_body(s_ref, a_ref, x_ref, wh_ref, bh_ref,
              w1_ref, b1_ref, w2_ref, b2_ref, o_ref):
    t = s_ref[...] + a_ref[0] + a_ref[1]
    t = _dot_t(t, wh_ref[...]) + bh_ref[...] + x_ref[...]
    t = _dot_t(t, w1_ref[...]) + b1_ref[...]
    o_ref[...] = _dot_t(t, w2_ref[...]) + b2_ref[...]


def _compute_out(s, aggr, x, Wh, bh, W1, b1, W2, b2):
    grid = (N // ROW_BLK,)
    row_spec = pl.BlockSpec((ROW_BLK, D), lambda i: (i, 0))
    a_spec = pl.BlockSpec((NC, ROW_BLK, D), lambda i: (0, i, 0))
    wdd_spec = pl.BlockSpec((D, D), lambda i: (0, 0))
    bd_spec = pl.BlockSpec((1, D), lambda i: (0, 0))
    w1_spec = pl.BlockSpec((2 * D, D), lambda i: (0, 0))
    b1_spec = pl.BlockSpec((1, 2 * D), lambda i: (0, 0))
    w2_spec = pl.BlockSpec((D, 2 * D), lambda i: (0, 0))
    return pl.pallas_call(
        _out_body,
        grid=grid,
        in_specs=[row_spec, a_spec, row_spec,
                  wdd_spec, bd_spec, w1_spec, b1_spec, w2_spec, bd_spec],
        out_specs=row_spec,
        out_shape=jax.ShapeDtypeStruct((N, D), jnp.float32),
    )(s, aggr, x, Wh, bh, W1, b1, W2, b2)


def kernel(x, edge_index, prompt, Wq, bq, Wk, bk, Wv, bv, Wh, bh,
           W1, b1, W2, b2):
    ei_p = _prep_edges(edge_index)
    h, s, zeros = _compute_h(x, prompt, Wq, bq.reshape(1, D),
                             Wk, bk.reshape(1, D), Wv, bv.reshape(1, D))
    aggr = _sc_seg_sum(h, ei_p, zeros)
    out = _compute_out(s, aggr, x, Wh, bh.reshape(1, D),
                       W1, b1.reshape(1, 2 * D), W2, b2.reshape(1, D))
    return out
